# baseline (speedup 1.0000x reference)
import functools

import jax
import jax.numpy as jnp
from jax.experimental import pallas as pl
from jax.experimental.pallas import tpu as pltpu


_N = 16


def _build_bd(row_ref, col_ref, ew_ref):
    f32 = jnp.float32
    r = row_ref[...]
    c = col_ref[...]
    w = ew_ref[...]
    E = r.shape[1]
    inv = (r == c)
    wo = jnp.where(inv, jnp.zeros_like(w), w)

    ni = jax.lax.broadcasted_iota(jnp.int32, (_N, E), 0)
    mr = (r == ni).astype(f32)
    mc = (c == ni).astype(f32)
    invf = inv.astype(f32)

    loop_contrib = jnp.sum(mr * (invf * w), axis=1, keepdims=True)
    has_loop = jnp.sum(mr * invf, axis=1, keepdims=True) > 0.0
    loop_w = jnp.where(has_loop, loop_contrib, jnp.ones_like(loop_contrib))

    deg = jnp.sum(mr * jnp.abs(wo), axis=1, keepdims=True) + jnp.abs(loop_w)
    dis = jnp.where(deg > 0.0, jax.lax.rsqrt(jnp.where(deg > 0.0, deg, jnp.ones_like(deg))), jnp.zeros_like(deg))

    p_un = jax.lax.dot_general(mc * wo, mr, (((1,), (1,)), ((), ())),
                               preferred_element_type=f32)
    i0 = jax.lax.broadcasted_iota(jnp.int32, (_N, _N), 0)
    i1 = jax.lax.broadcasted_iota(jnp.int32, (_N, _N), 1)
    eye = (i0 == i1).astype(f32)
    p_un = p_un + eye * loop_w
    dis_row = jnp.sum(eye * dis, axis=0, keepdims=True)
    P = (dis * p_un) * dis_row
    P2 = jax.lax.dot_general(P, P, (((1,), (0,)), ((), ())),
                             preferred_element_type=f32)

    imod = jax.lax.broadcasted_iota(jnp.int32, (128, _N), 0) % _N
    nn = jax.lax.broadcasted_iota(jnp.int32, (128, _N), 1)
    R = (imod == nn).astype(f32)
    t1 = jax.lax.dot_general(R, P2, (((1,), (0,)), ((), ())),
                             preferred_element_type=f32)
    tiled = jax.lax.dot_general(t1, R, (((1,), (1,)), ((), ())),
                                preferred_element_type=f32)
    ib = jax.lax.broadcasted_iota(jnp.int32, (128, 128), 0) // _N
    jb = jax.lax.broadcasted_iota(jnp.int32, (128, 128), 1) // _N
    return tiled * (ib == jb).astype(f32)


def _fused_body(row_ref, col_ref, ew_ref, x_ref, wlt_ref, bl_ref, wc_ref,
                bc_ref, wft_ref, bf_ref, o_ref, bd_s):
    f32 = jnp.float32

    @pl.when(pl.program_id(0) == 0)
    def _():
        bd_s[...] = _build_bd(row_ref, col_ref, ew_ref)

    bd = bd_s[...]
    Rr = x_ref.shape[0]
    wlt = wlt_ref[...]
    bl = bl_ref[...]

    imod = jax.lax.broadcasted_iota(jnp.int32, (128, _N), 0) % _N
    nn = jax.lax.broadcasted_iota(jnp.int32, (128, _N), 1)
    Rm = (imod == nn).astype(f32)
    wcol = jax.lax.dot_general(Rm, wc_ref[...], (((1,), (0,)), ((), ())),
                               preferred_element_type=f32)
    si = jax.lax.broadcasted_iota(jnp.int32, (128 // _N, 128), 0)
    rj = jax.lax.broadcasted_iota(jnp.int32, (128 // _N, 128), 1) // _N
    M8 = (si == rj).astype(f32)

    zs = []
    for k in range(Rr // 128):
        xk = x_ref[k * 128:(k + 1) * 128, :]
        y = jax.lax.dot_general(bd, xk, (((1,), (0,)), ((), ())),
                                preferred_element_type=f32)
        h = jax.lax.dot_general(y, wlt, (((1,), (0,)), ((), ())),
                                preferred_element_type=f32)
        h = jnp.maximum(h + bl, 0.0)
        zs.append(jax.lax.dot_general(M8, h * wcol, (((1,), (0,)), ((), ())),
                                      preferred_element_type=f32))
    z = jnp.concatenate(zs, axis=0) if len(zs) > 1 else zs[0]
    z = z + bc_ref[...]
    o = jax.lax.dot_general(z, wft_ref[...], (((1,), (0,)), ((), ())),
                            preferred_element_type=f32)
    o_ref[...] = o + bf_ref[...]


@functools.partial(jax.jit, static_argnames=())
def kernel(x, edge_weight, W_lin, b_lin, W_conv2, b_conv2, W_fc, b_fc, edge_idx):
    B, N, F = x.shape
    H = W_lin.shape[0]
    C = W_fc.shape[0]
    E = edge_idx.shape[1]
    assert N == _N

    row = edge_idx[0:1, :]
    col = edge_idx[1:2, :]
    ew = edge_weight.reshape(1, E)

    SAMP = 128
    Rrows = SAMP * N
    M = B * N
    grid = (M // Rrows,)

    x2 = x.reshape(M, F)
    wlt = W_lin.T
    bl = b_lin.reshape(1, H)
    wc = W_conv2.reshape(N, 1)
    bc = b_conv2.reshape(1, 1)
    wft = W_fc.T
    bf = b_fc.reshape(1, C)

    out = pl.pallas_call(
        _fused_body,
        grid=grid,
        in_specs=[
            pl.BlockSpec((1, E), lambda i: (0, 0)),
            pl.BlockSpec((1, E), lambda i: (0, 0)),
            pl.BlockSpec((1, E), lambda i: (0, 0)),
            pl.BlockSpec((Rrows, F), lambda i: (i, 0)),
            pl.BlockSpec((F, H), lambda i: (0, 0)),
            pl.BlockSpec((1, H), lambda i: (0, 0)),
            pl.BlockSpec((N, 1), lambda i: (0, 0)),
            pl.BlockSpec((1, 1), lambda i: (0, 0)),
            pl.BlockSpec((H, C), lambda i: (0, 0)),
            pl.BlockSpec((1, C), lambda i: (0, 0)),
        ],
        out_specs=pl.BlockSpec((SAMP, C), lambda i: (i, 0)),
        out_shape=jax.ShapeDtypeStruct((B, C), jnp.float32),
        scratch_shapes=[pltpu.VMEM((128, 128), jnp.float32)],
    )(row, col, ew, x2, wlt, bl, wc, bc, wft, bf)
    return out


# speedup vs baseline: 2.4477x; 2.4477x over previous
import functools

import jax
import jax.numpy as jnp
from jax.experimental import pallas as pl
from jax.experimental.pallas import tpu as pltpu


_N = 16


def _build_bd(row_ref, col_ref, ew_ref):
    f32 = jnp.float32
    r = row_ref[...]
    c = col_ref[...]
    w = ew_ref[...]
    E = r.shape[1]
    inv = (r == c)
    wo = jnp.where(inv, jnp.zeros_like(w), w)

    ni = jax.lax.broadcasted_iota(jnp.int32, (_N, E), 0)
    mr = (r == ni).astype(f32)
    mc = (c == ni).astype(f32)
    invf = inv.astype(f32)

    loop_contrib = jnp.sum(mr * (invf * w), axis=1, keepdims=True)
    has_loop = jnp.sum(mr * invf, axis=1, keepdims=True) > 0.0
    loop_w = jnp.where(has_loop, loop_contrib, jnp.ones_like(loop_contrib))

    deg = jnp.sum(mr * jnp.abs(wo), axis=1, keepdims=True) + jnp.abs(loop_w)
    dis = jnp.where(deg > 0.0, jax.lax.rsqrt(jnp.where(deg > 0.0, deg, jnp.ones_like(deg))), jnp.zeros_like(deg))

    p_un = jax.lax.dot_general(mc * wo, mr, (((1,), (1,)), ((), ())),
                               preferred_element_type=f32)
    i0 = jax.lax.broadcasted_iota(jnp.int32, (_N, _N), 0)
    i1 = jax.lax.broadcasted_iota(jnp.int32, (_N, _N), 1)
    eye = (i0 == i1).astype(f32)
    p_un = p_un + eye * loop_w
    dis_row = jnp.sum(eye * dis, axis=0, keepdims=True)
    P = (dis * p_un) * dis_row
    P2 = jax.lax.dot_general(P, P, (((1,), (0,)), ((), ())),
                             preferred_element_type=f32)

    imod = jax.lax.broadcasted_iota(jnp.int32, (128, _N), 0) % _N
    nn = jax.lax.broadcasted_iota(jnp.int32, (128, _N), 1)
    R = (imod == nn).astype(f32)
    t1 = jax.lax.dot_general(R, P2, (((1,), (0,)), ((), ())),
                             preferred_element_type=f32)
    tiled = jax.lax.dot_general(t1, R, (((1,), (1,)), ((), ())),
                                preferred_element_type=f32)
    ib = jax.lax.broadcasted_iota(jnp.int32, (128, 128), 0) // _N
    jb = jax.lax.broadcasted_iota(jnp.int32, (128, 128), 1) // _N
    return tiled * (ib == jb).astype(f32)


def _fused_body(row_ref, col_ref, ew_ref, x_ref, wlt_ref, bl_ref, wc_ref,
                bc_ref, wft_ref, bf_ref, o_ref, bd_s):
    f32 = jnp.float32

    @pl.when(pl.program_id(0) == 0)
    def _():
        bd_s[...] = _build_bd(row_ref, col_ref, ew_ref)

    bd = bd_s[...]
    Rr = x_ref.shape[0]
    S = Rr // _N
    wlt = wlt_ref[...]
    bl = bl_ref[...]

    ys = []
    for k in range(Rr // 128):
        xk = x_ref[k * 128:(k + 1) * 128, :]
        ys.append(jax.lax.dot_general(bd, xk, (((1,), (0,)), ((), ())),
                                      preferred_element_type=f32))
    y = jnp.concatenate(ys, axis=0) if len(ys) > 1 else ys[0]
    h = jax.lax.dot_general(y, wlt, (((1,), (0,)), ((), ())),
                            preferred_element_type=f32)
    h = jnp.maximum(h + bl, 0.0)

    h3 = h.reshape(S, _N, h.shape[1])
    wc3 = wc_ref[...].reshape(1, _N, 1)
    z = jnp.sum(h3 * wc3, axis=1)
    z = z + bc_ref[...]
    o = jax.lax.dot_general(z, wft_ref[...], (((1,), (0,)), ((), ())),
                            preferred_element_type=f32)
    o_ref[...] = o + bf_ref[...]


@functools.partial(jax.jit, static_argnames=())
def kernel(x, edge_weight, W_lin, b_lin, W_conv2, b_conv2, W_fc, b_fc, edge_idx):
    B, N, F = x.shape
    H = W_lin.shape[0]
    C = W_fc.shape[0]
    E = edge_idx.shape[1]
    assert N == _N

    row = edge_idx[0:1, :]
    col = edge_idx[1:2, :]
    ew = edge_weight.reshape(1, E)

    SAMP = 128
    Rrows = SAMP * N
    M = B * N
    grid = (M // Rrows,)

    x2 = x.reshape(M, F)
    wlt = W_lin.T
    bl = b_lin.reshape(1, H)
    wc = W_conv2.reshape(N, 1)
    bc = b_conv2.reshape(1, 1)
    wft = W_fc.T
    bf = b_fc.reshape(1, C)

    out = pl.pallas_call(
        _fused_body,
        grid=grid,
        in_specs=[
            pl.BlockSpec((1, E), lambda i: (0, 0)),
            pl.BlockSpec((1, E), lambda i: (0, 0)),
            pl.BlockSpec((1, E), lambda i: (0, 0)),
            pl.BlockSpec((Rrows, F), lambda i: (i, 0)),
            pl.BlockSpec((F, H), lambda i: (0, 0)),
            pl.BlockSpec((1, H), lambda i: (0, 0)),
            pl.BlockSpec((N, 1), lambda i: (0, 0)),
            pl.BlockSpec((1, 1), lambda i: (0, 0)),
            pl.BlockSpec((H, C), lambda i: (0, 0)),
            pl.BlockSpec((1, C), lambda i: (0, 0)),
        ],
        out_specs=pl.BlockSpec((SAMP, C), lambda i: (i, 0)),
        out_shape=jax.ShapeDtypeStruct((B, C), jnp.float32),
        scratch_shapes=[pltpu.VMEM((128, 128), jnp.float32)],
    )(row, col, ew, x2, wlt, bl, wc, bc, wft, bf)
    return out
